# baseline (speedup 1.0000x reference)
import functools

import jax
import jax.numpy as jnp
from jax import lax
from jax.experimental import pallas as pl
from jax.experimental.pallas import tpu as pltpu
from jax.experimental.pallas import tpu_sc as plsc

EMB = 128
SEQ = 200
C0, C1 = 128, 72
HB = 8
QK = 4 * HB
G = 2


def _gather_body(ts_hbm, table_hbm, out_hbm, table_sp, idx_a, idx_b,
                 pb0, pb1, g0, g1, g2, g3, s0, s1,
                 sem_ia, sem_ib, *, nc, steps_per_w):
    sid = lax.axis_index("s")
    wid = sid * nc + lax.axis_index("c")
    row0 = wid * steps_per_w * (2 * HB)
    pairs = (pb0, pb1)
    gsems = (g0, g1, g2, g3)
    ssems = (s0, s1)

    pl.when(sid == 0)(lambda: pltpu.sync_copy(table_hbm, table_sp))
    plsc.subcore_barrier()

    def slot_ref(q):
        pb = pairs[(q // 2) % 2]
        if q % 2 == 0:
            return pb.at[pl.ds(0, C0)]
        return pb.at[pl.ds(C0, C1)]

    def drain_store(p):
        pltpu.make_async_copy(pairs[p], out_hbm.at[pl.ds(0, SEQ)],
                              ssems[p]).wait()

    def idx_ref(q):
        q = q % QK
        buf, lrow = (idx_a, (q // 2) % HB) if (q // 2) % (2 * HB) < HB else \
                    (idx_b, (q // 2) % HB)
        if q % 2 == 0:
            return buf.at[lrow, pl.ds(0, C0)]
        return buf.at[lrow, pl.ds(C0, C1)]

    def table_for(q):
        return table_hbm if (q % QK) // 2 == 2 * HB - 1 else table_sp

    def fire_gather(q):
        pltpu.async_copy(table_for(q).at[idx_ref(q)], slot_ref(q),
                         gsems[q % 4])

    def wait_gather(q):
        pltpu.make_async_copy(table_for(q).at[idx_a.at[0, pl.ds(0, C0
                                                                if q % 2 == 0
                                                                else C1)]],
                              slot_ref(q), gsems[q % 4]).wait()

    def drain_idx(buf, sem):
        pltpu.make_async_copy(ts_hbm.at[pl.ds(0, HB)], buf, sem).wait()

    pltpu.sync_copy(ts_hbm.at[pl.ds(row0, HB)], idx_a)
    for j in range(G):
        fire_gather(j)

    def step(t, carry):
        r = row0 + t * 2 * HB
        for j in range(QK):
            if j == 0:
                pltpu.async_copy(ts_hbm.at[pl.ds(r + HB, HB)], idx_b, sem_ib)
            elif j == 2 * HB - G - 1:
                drain_idx(idx_b, sem_ib)
            elif j == 2 * HB:
                nxt = jnp.minimum(t + 1, steps_per_w - 1)
                pltpu.async_copy(ts_hbm.at[pl.ds(row0 + nxt * 2 * HB, HB)],
                                 idx_a, sem_ia)
            elif j == QK - G - 1:
                drain_idx(idx_a, sem_ia)
            if (j + G) % 2 == 0:
                p = ((j + G) // 2) % 2
                if j < G:
                    pl.when(t > 0)(lambda p=p: drain_store(p))
                else:
                    drain_store(p)
            fire_gather(j + G)
            wait_gather(j)
            if j % 2 == 1:
                p = (j // 2) % 2
                off = (r + j // 2) * SEQ
                pltpu.async_copy(pairs[p], out_hbm.at[pl.ds(off, SEQ)],
                                 ssems[p])
        return carry

    lax.fori_loop(0, steps_per_w, step, 0)
    for j in range(G):
        wait_gather(j)
    drain_store(((QK - 1) // 2) % 2)


def kernel(timestamps, table):
    n, s = timestamps.shape
    assert s == SEQ
    b = n * s

    info = plsc.get_sparse_core_info()
    nc, ns = info.num_cores, info.num_subcores
    nw = nc * ns
    assert n % (nw * 2 * HB) == 0
    steps_per_w = n // (nw * 2 * HB)

    mesh = plsc.VectorSubcoreMesh(core_axis_name="c", subcore_axis_name="s")
    out = pl.kernel(
        functools.partial(_gather_body, nc=nc, steps_per_w=steps_per_w),
        out_type=jax.ShapeDtypeStruct((b, EMB), jnp.float32),
        mesh=mesh,
        scratch_types=[pltpu.VMEM_SHARED(table.shape, jnp.float32),
                       pltpu.VMEM((HB, SEQ), jnp.int32),
                       pltpu.VMEM((HB, SEQ), jnp.int32),
                       pltpu.VMEM((SEQ, EMB), jnp.float32),
                       pltpu.VMEM((SEQ, EMB), jnp.float32)]
        + [pltpu.SemaphoreType.DMA for _ in range(8)],
    )(timestamps.astype(jnp.int32), table)
    return out.reshape(n, s, EMB)


# speedup vs baseline: 1.0340x; 1.0340x over previous
import functools

import jax
import jax.numpy as jnp
from jax import lax
from jax.experimental import pallas as pl
from jax.experimental.pallas import tpu as pltpu
from jax.experimental.pallas import tpu_sc as plsc

EMB = 128
SEQ = 200
C0, C1 = 128, 72
HB = 8
QK = 4 * HB
G = 2


def _gather_body(ts_hbm, table_hbm, out_hbm, table_sp, idx_a, idx_b,
                 pb0, pb1, g0, g1, g2, g3, s0, s1,
                 sem_ia, sem_ib, *, nc, steps_per_w):
    sid = lax.axis_index("s")
    wid = sid * nc + lax.axis_index("c")
    row0 = wid * steps_per_w * (2 * HB)
    pairs = (pb0, pb1)
    gsems = (g0, g1, g2, g3)
    ssems = (s0, s1)

    pl.when(sid == 0)(lambda: pltpu.sync_copy(table_hbm, table_sp))
    plsc.subcore_barrier()

    def slot_ref(q):
        pb = pairs[(q // 2) % 2]
        if q % 2 == 0:
            return pb.at[pl.ds(0, C0)]
        return pb.at[pl.ds(C0, C1)]

    def drain_store(p):
        pltpu.make_async_copy(pairs[p], out_hbm.at[pl.ds(0, SEQ)],
                              ssems[p]).wait()

    def idx_ref(q):
        q = q % QK
        buf, lrow = (idx_a, (q // 2) % HB) if (q // 2) % (2 * HB) < HB else \
                    (idx_b, (q // 2) % HB)
        if q % 2 == 0:
            return buf.at[lrow, pl.ds(0, C0)]
        return buf.at[lrow, pl.ds(C0, C1)]

    def fire_gather(q):
        pltpu.async_copy(table_sp.at[idx_ref(q)], slot_ref(q), gsems[q % 4])

    def wait_gather(q):
        pltpu.make_async_copy(table_sp.at[idx_a.at[0, pl.ds(0, C0 if q % 2 == 0
                                                            else C1)]],
                              slot_ref(q), gsems[q % 4]).wait()

    def drain_idx(buf, sem):
        pltpu.make_async_copy(ts_hbm.at[pl.ds(0, HB)], buf, sem).wait()

    pltpu.sync_copy(ts_hbm.at[pl.ds(row0, HB)], idx_a)
    for j in range(G):
        fire_gather(j)

    def step(t, carry):
        r = row0 + t * 2 * HB
        for j in range(QK):
            if j == 0:
                pltpu.async_copy(ts_hbm.at[pl.ds(r + HB, HB)], idx_b, sem_ib)
            elif j == 2 * HB - G - 1:
                drain_idx(idx_b, sem_ib)
            elif j == 2 * HB:
                nxt = jnp.minimum(t + 1, steps_per_w - 1)
                pltpu.async_copy(ts_hbm.at[pl.ds(row0 + nxt * 2 * HB, HB)],
                                 idx_a, sem_ia)
            elif j == QK - G - 1:
                drain_idx(idx_a, sem_ia)
            if (j + G) % 2 == 0:
                p = ((j + G) // 2) % 2
                if j < G:
                    pl.when(t > 0)(lambda p=p: drain_store(p))
                else:
                    drain_store(p)
            fire_gather(j + G)
            wait_gather(j)
            if j % 2 == 1:
                p = (j // 2) % 2
                off = (r + j // 2) * SEQ
                pltpu.async_copy(pairs[p], out_hbm.at[pl.ds(off, SEQ)],
                                 ssems[p])
        return carry

    lax.fori_loop(0, steps_per_w, step, 0)
    for j in range(G):
        wait_gather(j)
    drain_store(((QK - 1) // 2) % 2)


def kernel(timestamps, table):
    n, s = timestamps.shape
    assert s == SEQ
    b = n * s

    info = plsc.get_sparse_core_info()
    nc, ns = info.num_cores, info.num_subcores
    nw = nc * ns
    assert n % (nw * 2 * HB) == 0
    steps_per_w = n // (nw * 2 * HB)

    mesh = plsc.VectorSubcoreMesh(core_axis_name="c", subcore_axis_name="s")
    out = pl.kernel(
        functools.partial(_gather_body, nc=nc, steps_per_w=steps_per_w),
        out_type=jax.ShapeDtypeStruct((b, EMB), jnp.float32),
        mesh=mesh,
        scratch_types=[pltpu.VMEM_SHARED(table.shape, jnp.float32),
                       pltpu.VMEM((HB, SEQ), jnp.int32),
                       pltpu.VMEM((HB, SEQ), jnp.int32),
                       pltpu.VMEM((SEQ, EMB), jnp.float32),
                       pltpu.VMEM((SEQ, EMB), jnp.float32)]
        + [pltpu.SemaphoreType.DMA for _ in range(8)],
    )(timestamps.astype(jnp.int32), table)
    return out.reshape(n, s, EMB)
